# baseline (speedup 1.0000x reference)
import functools

import jax
import jax.numpy as jnp
from jax import lax
from jax.experimental import pallas as pl
from jax.experimental.pallas import tpu as pltpu
from jax.experimental.pallas import tpu_sc as plsc

B, T, D, K = 128, 8192, 64, 32
RB = 16
CH = 8
PH = 8
BP = B // PH


def _topk_body(s_ref, vals_ref, idx_ref):
    sb = s_ref[...]
    iota = lax.broadcasted_iota(jnp.int32, (RB, T), 1)
    kiota = lax.broadcasted_iota(jnp.int32, (RB, K), 1)
    neg = jnp.float32(-jnp.inf)

    def body(k, carry):
        sb, vals, idxs = carry
        m = jnp.max(sb, axis=1, keepdims=True)
        im = jnp.min(jnp.where(sb == m, iota, T), axis=1, keepdims=True)
        vals = jnp.where(kiota == k, m, vals)
        idxs = jnp.where(kiota == k, im, idxs)
        sb = jnp.where(iota == im, neg, sb)
        return sb, vals, idxs

    init = (sb, jnp.zeros((RB, K), jnp.float32), jnp.zeros((RB, K), jnp.int32))
    _, vals, idxs = lax.fori_loop(0, K, body, init)
    vals_ref[...] = vals
    idx_ref[...] = idxs


_topk = pl.pallas_call(
    _topk_body,
    grid=(BP // RB,),
    in_specs=[pl.BlockSpec((RB, T), lambda i: (i, 0))],
    out_specs=[
        pl.BlockSpec((RB, K), lambda i: (i, 0)),
        pl.BlockSpec((RB, K), lambda i: (i, 0)),
    ],
    out_shape=[
        jax.ShapeDtypeStruct((BP, K), jnp.float32),
        jax.ShapeDtypeStruct((BP, K), jnp.int32),
    ],
)


def _make_gather(b_off):
    info = plsc.get_sparse_core_info()
    nw = info.num_cores * info.num_subcores
    n = BP * K
    per_w = n // nw
    mesh = plsc.VectorSubcoreMesh(core_axis_name="c", subcore_axis_name="s")

    @functools.partial(
        pl.kernel,
        out_type=jax.ShapeDtypeStruct((n * D,), jnp.float32),
        mesh=mesh,
        compiler_params=pltpu.CompilerParams(needs_layout_passes=False),
        scratch_types=[
            pltpu.VMEM((per_w + 16,), jnp.int32),
            pltpu.VMEM((CH, D, 128), jnp.float32),
            pltpu.VMEM((per_w * D,), jnp.float32),
            pltpu.SemaphoreType.DMA,
        ],
    )
    def gather(xt_hbm, idx_hbm, out_hbm, idx_v, blocks, rows, sem):
        wid = lax.axis_index("s") * info.num_cores + lax.axis_index("c")
        base = wid * per_w
        lane = lax.iota(jnp.int32, 16)
        pltpu.sync_copy(idx_hbm.at[pl.ds(base, per_w)], idx_v.at[pl.ds(0, per_w)])

        def block_copy(t, i, u):
            b = b_off + (base + i) // K
            off = pl.multiple_of((t // 128) * 128, 128)
            return pltpu.make_async_copy(
                xt_hbm.at[b, :, pl.ds(off, 128)], blocks.at[u], sem
            )

        def chunk(c, _):
            i0 = c * CH
            iv = idx_v[pl.ds(pl.multiple_of(i0, CH), 16)]
            ts = [jnp.max(jnp.where(lane == u, iv, 0)) for u in range(CH)]
            for u in range(CH):
                block_copy(ts[u], i0 + u, u).start()
            for u in range(CH):
                block_copy(ts[u], i0 + u, u).wait()
            for u in range(CH):
                i = i0 + u
                j = jnp.full((16,), ts[u] % 128, jnp.int32)
                uu = jnp.full((16,), u, jnp.int32)
                for g in range(D // 16):
                    v = plsc.load_gather(blocks, [uu, g * 16 + lane, j])
                    plsc.store_scatter(rows, [i * D + g * 16 + lane], v)
            return 0

        lax.fori_loop(0, per_w // CH, chunk, 0)
        pltpu.sync_copy(rows, out_hbm.at[pl.ds(base * D, per_w * D)])

    return gather


_gathers = [_make_gather(h * BP) for h in range(PH)]


@jax.jit
def kernel(s, x):
    s2 = s.reshape(B, T)
    xt = jnp.transpose(x, (0, 2, 1))
    outs = []
    for h in range(PH):
        vals, idx = _topk(lax.slice(s2, (h * BP, 0), ((h + 1) * BP, T)))
        rows = _gathers[h](xt, idx.reshape(BP * K))
        outs.append(
            jnp.concatenate(
                [vals.reshape(BP, K, 1), rows.reshape(BP, K, D)], axis=-1
            )
        )
    return jnp.concatenate(outs, axis=0)


# speedup vs baseline: 1.1948x; 1.1948x over previous
import functools

import jax
import jax.numpy as jnp
from jax import lax
from jax.experimental import pallas as pl
from jax.experimental.pallas import tpu as pltpu
from jax.experimental.pallas import tpu_sc as plsc

B, T, D, K = 128, 8192, 64, 32
RB = 16
CH = 8
PH = 4
BP = B // PH


def _topk_body(s_ref, vals_ref, idx_ref):
    sb = s_ref[...]
    iota = lax.broadcasted_iota(jnp.int32, (RB, T), 1)
    kiota = lax.broadcasted_iota(jnp.int32, (RB, K), 1)
    neg = jnp.float32(-jnp.inf)

    def body(k, carry):
        sb, vals, idxs = carry
        m = jnp.max(sb, axis=1, keepdims=True)
        im = jnp.min(jnp.where(sb == m, iota, T), axis=1, keepdims=True)
        vals = jnp.where(kiota == k, m, vals)
        idxs = jnp.where(kiota == k, im, idxs)
        sb = jnp.where(iota == im, neg, sb)
        return sb, vals, idxs

    carry = (sb, jnp.zeros((RB, K), jnp.float32), jnp.zeros((RB, K), jnp.int32))
    for k in range(K):
        carry = body(k, carry)
    _, vals, idxs = carry
    vals_ref[...] = vals
    idx_ref[...] = idxs


_topk = pl.pallas_call(
    _topk_body,
    grid=(BP // RB,),
    in_specs=[pl.BlockSpec((RB, T), lambda i: (i, 0))],
    out_specs=[
        pl.BlockSpec((RB, K), lambda i: (i, 0)),
        pl.BlockSpec((RB, K), lambda i: (i, 0)),
    ],
    out_shape=[
        jax.ShapeDtypeStruct((BP, K), jnp.float32),
        jax.ShapeDtypeStruct((BP, K), jnp.int32),
    ],
)


def _make_gather(b_off):
    info = plsc.get_sparse_core_info()
    nw = info.num_cores * info.num_subcores
    n = BP * K
    per_w = n // nw
    mesh = plsc.VectorSubcoreMesh(core_axis_name="c", subcore_axis_name="s")

    @functools.partial(
        pl.kernel,
        out_type=jax.ShapeDtypeStruct((n * D,), jnp.float32),
        mesh=mesh,
        compiler_params=pltpu.CompilerParams(needs_layout_passes=False),
        scratch_types=[
            pltpu.VMEM((per_w + 16,), jnp.int32),
            pltpu.VMEM((CH, D, 128), jnp.float32),
            pltpu.VMEM((per_w * D,), jnp.float32),
            pltpu.SemaphoreType.DMA,
        ],
    )
    def gather(xt_hbm, idx_hbm, out_hbm, idx_v, blocks, rows, sem):
        wid = lax.axis_index("s") * info.num_cores + lax.axis_index("c")
        base = wid * per_w
        lane = lax.iota(jnp.int32, 16)
        pltpu.sync_copy(idx_hbm.at[pl.ds(base, per_w)], idx_v.at[pl.ds(0, per_w)])

        def block_copy(t, i, u):
            b = b_off + (base + i) // K
            off = pl.multiple_of((t // 128) * 128, 128)
            return pltpu.make_async_copy(
                xt_hbm.at[b, :, pl.ds(off, 128)], blocks.at[u], sem
            )

        def chunk(c, _):
            i0 = c * CH
            iv = idx_v[pl.ds(pl.multiple_of(i0, CH), 16)]
            ts = [jnp.max(jnp.where(lane == u, iv, 0)) for u in range(CH)]
            for u in range(CH):
                block_copy(ts[u], i0 + u, u).start()
            for u in range(CH):
                block_copy(ts[u], i0 + u, u).wait()
            for u in range(CH):
                i = i0 + u
                j = jnp.full((16,), ts[u] % 128, jnp.int32)
                uu = jnp.full((16,), u, jnp.int32)
                for g in range(D // 16):
                    v = plsc.load_gather(blocks, [uu, g * 16 + lane, j])
                    plsc.store_scatter(rows, [i * D + g * 16 + lane], v)
            return 0

        lax.fori_loop(0, per_w // CH, chunk, 0)
        pltpu.sync_copy(rows, out_hbm.at[pl.ds(base * D, per_w * D)])

    return gather


_gathers = [_make_gather(h * BP) for h in range(PH)]


@jax.jit
def kernel(s, x):
    s2 = s.reshape(B, T)
    xt = jnp.transpose(x, (0, 2, 1))
    outs = []
    for h in range(PH):
        vals, idx = _topk(lax.slice(s2, (h * BP, 0), ((h + 1) * BP, T)))
        rows = _gathers[h](xt, idx.reshape(BP * K))
        outs.append(
            jnp.concatenate(
                [vals.reshape(BP, K, 1), rows.reshape(BP, K, D)], axis=-1
            )
        )
    return jnp.concatenate(outs, axis=0)


# speedup vs baseline: 1.3344x; 1.1169x over previous
import functools

import jax
import jax.numpy as jnp
from jax import lax
from jax.experimental import pallas as pl
from jax.experimental.pallas import tpu as pltpu
from jax.experimental.pallas import tpu_sc as plsc

B, T, D, K = 128, 8192, 64, 32
RB = 32
CH = 8
PH = 4
BP = B // PH


def _topk_body(s_ref, vals_ref, idx_ref):
    sb = s_ref[...]
    iota = lax.broadcasted_iota(jnp.int32, (RB, T), 1)
    kiota = lax.broadcasted_iota(jnp.int32, (RB, K), 1)
    neg = jnp.float32(-jnp.inf)

    def body(k, carry):
        sb, vals, idxs = carry
        m = jnp.max(sb, axis=1, keepdims=True)
        im = jnp.min(jnp.where(sb == m, iota, T), axis=1, keepdims=True)
        vals = jnp.where(kiota == k, m, vals)
        idxs = jnp.where(kiota == k, im, idxs)
        sb = jnp.where(iota == im, neg, sb)
        return sb, vals, idxs

    carry = (sb, jnp.zeros((RB, K), jnp.float32), jnp.zeros((RB, K), jnp.int32))
    for k in range(K):
        carry = body(k, carry)
    _, vals, idxs = carry
    vals_ref[...] = vals
    idx_ref[...] = idxs


_topk = pl.pallas_call(
    _topk_body,
    grid=(BP // RB,),
    in_specs=[pl.BlockSpec((RB, T), lambda i: (i, 0))],
    out_specs=[
        pl.BlockSpec((RB, K), lambda i: (i, 0)),
        pl.BlockSpec((RB, K), lambda i: (i, 0)),
    ],
    out_shape=[
        jax.ShapeDtypeStruct((BP, K), jnp.float32),
        jax.ShapeDtypeStruct((BP, K), jnp.int32),
    ],
)


def _make_gather(b_off):
    info = plsc.get_sparse_core_info()
    nw = info.num_cores * info.num_subcores
    n = BP * K
    per_w = n // nw
    mesh = plsc.VectorSubcoreMesh(core_axis_name="c", subcore_axis_name="s")

    @functools.partial(
        pl.kernel,
        out_type=jax.ShapeDtypeStruct((n * D,), jnp.float32),
        mesh=mesh,
        compiler_params=pltpu.CompilerParams(needs_layout_passes=False),
        scratch_types=[
            pltpu.VMEM((per_w + 16,), jnp.int32),
            pltpu.VMEM((CH, D, 128), jnp.float32),
            pltpu.VMEM((per_w * D,), jnp.float32),
            pltpu.SemaphoreType.DMA,
        ],
    )
    def gather(xt_hbm, idx_hbm, out_hbm, idx_v, blocks, rows, sem):
        wid = lax.axis_index("s") * info.num_cores + lax.axis_index("c")
        base = wid * per_w
        lane = lax.iota(jnp.int32, 16)
        pltpu.sync_copy(idx_hbm.at[pl.ds(base, per_w)], idx_v.at[pl.ds(0, per_w)])

        def block_copy(t, i, u):
            b = b_off + (base + i) // K
            off = pl.multiple_of((t // 128) * 128, 128)
            return pltpu.make_async_copy(
                xt_hbm.at[b, :, pl.ds(off, 128)], blocks.at[u], sem
            )

        def chunk(c, _):
            i0 = c * CH
            iv = idx_v[pl.ds(pl.multiple_of(i0, CH), 16)]
            ts = [jnp.max(jnp.where(lane == u, iv, 0)) for u in range(CH)]
            for u in range(CH):
                block_copy(ts[u], i0 + u, u).start()
            for u in range(CH):
                block_copy(ts[u], i0 + u, u).wait()
            for u in range(CH):
                i = i0 + u
                j = jnp.full((16,), ts[u] % 128, jnp.int32)
                uu = jnp.full((16,), u, jnp.int32)
                for g in range(D // 16):
                    v = plsc.load_gather(blocks, [uu, g * 16 + lane, j])
                    plsc.store_scatter(rows, [i * D + g * 16 + lane], v)
            return 0

        lax.fori_loop(0, per_w // CH, chunk, 0)
        pltpu.sync_copy(rows, out_hbm.at[pl.ds(base * D, per_w * D)])

    return gather


_gathers = [_make_gather(h * BP) for h in range(PH)]


@jax.jit
def kernel(s, x):
    s2 = s.reshape(B, T)
    xt = jnp.transpose(x, (0, 2, 1))
    outs = []
    for h in range(PH):
        vals, idx = _topk(lax.slice(s2, (h * BP, 0), ((h + 1) * BP, T)))
        rows = _gathers[h](xt, idx.reshape(BP * K))
        outs.append(
            jnp.concatenate(
                [vals.reshape(BP, K, 1), rows.reshape(BP, K, D)], axis=-1
            )
        )
    return jnp.concatenate(outs, axis=0)


# speedup vs baseline: 1.3521x; 1.0132x over previous
import functools

import jax
import jax.numpy as jnp
from jax import lax
from jax.experimental import pallas as pl
from jax.experimental.pallas import tpu as pltpu
from jax.experimental.pallas import tpu_sc as plsc

B, T, D, K = 128, 8192, 64, 32
RB = 64
CH = 8
PH = 2
BP = B // PH


def _topk_body(s_ref, vals_ref, idx_ref):
    sb = s_ref[...]
    iota = lax.broadcasted_iota(jnp.int32, (RB, T), 1)
    kiota = lax.broadcasted_iota(jnp.int32, (RB, K), 1)
    neg = jnp.float32(-jnp.inf)

    def body(k, carry):
        sb, vals, idxs = carry
        m = jnp.max(sb, axis=1, keepdims=True)
        im = jnp.min(jnp.where(sb == m, iota, T), axis=1, keepdims=True)
        vals = jnp.where(kiota == k, m, vals)
        idxs = jnp.where(kiota == k, im, idxs)
        sb = jnp.where(iota == im, neg, sb)
        return sb, vals, idxs

    carry = (sb, jnp.zeros((RB, K), jnp.float32), jnp.zeros((RB, K), jnp.int32))
    for k in range(K):
        carry = body(k, carry)
    _, vals, idxs = carry
    vals_ref[...] = vals
    idx_ref[...] = idxs


_topk = pl.pallas_call(
    _topk_body,
    grid=(BP // RB,),
    in_specs=[pl.BlockSpec((RB, T), lambda i: (i, 0))],
    out_specs=[
        pl.BlockSpec((RB, K), lambda i: (i, 0)),
        pl.BlockSpec((RB, K), lambda i: (i, 0)),
    ],
    out_shape=[
        jax.ShapeDtypeStruct((BP, K), jnp.float32),
        jax.ShapeDtypeStruct((BP, K), jnp.int32),
    ],
)


def _make_gather(b_off):
    info = plsc.get_sparse_core_info()
    nw = info.num_cores * info.num_subcores
    n = BP * K
    per_w = n // nw
    mesh = plsc.VectorSubcoreMesh(core_axis_name="c", subcore_axis_name="s")

    @functools.partial(
        pl.kernel,
        out_type=jax.ShapeDtypeStruct((n * D,), jnp.float32),
        mesh=mesh,
        compiler_params=pltpu.CompilerParams(needs_layout_passes=False),
        scratch_types=[
            pltpu.VMEM((per_w + 16,), jnp.int32),
            pltpu.VMEM((CH, D, 128), jnp.float32),
            pltpu.VMEM((per_w * D,), jnp.float32),
            pltpu.SemaphoreType.DMA,
        ],
    )
    def gather(xt_hbm, idx_hbm, out_hbm, idx_v, blocks, rows, sem):
        wid = lax.axis_index("s") * info.num_cores + lax.axis_index("c")
        base = wid * per_w
        lane = lax.iota(jnp.int32, 16)
        pltpu.sync_copy(idx_hbm.at[pl.ds(base, per_w)], idx_v.at[pl.ds(0, per_w)])

        def block_copy(t, i, u):
            b = b_off + (base + i) // K
            off = pl.multiple_of((t // 128) * 128, 128)
            return pltpu.make_async_copy(
                xt_hbm.at[b, :, pl.ds(off, 128)], blocks.at[u], sem
            )

        def chunk(c, _):
            i0 = c * CH
            iv = idx_v[pl.ds(pl.multiple_of(i0, CH), 16)]
            ts = [jnp.max(jnp.where(lane == u, iv, 0)) for u in range(CH)]
            for u in range(CH):
                block_copy(ts[u], i0 + u, u).start()
            for u in range(CH):
                block_copy(ts[u], i0 + u, u).wait()
            for u in range(CH):
                i = i0 + u
                j = jnp.full((16,), ts[u] % 128, jnp.int32)
                uu = jnp.full((16,), u, jnp.int32)
                for g in range(D // 16):
                    v = plsc.load_gather(blocks, [uu, g * 16 + lane, j])
                    plsc.store_scatter(rows, [i * D + g * 16 + lane], v)
            return 0

        lax.fori_loop(0, per_w // CH, chunk, 0)
        pltpu.sync_copy(rows, out_hbm.at[pl.ds(base * D, per_w * D)])

    return gather


_gathers = [_make_gather(h * BP) for h in range(PH)]


@jax.jit
def kernel(s, x):
    s2 = s.reshape(B, T)
    xt = jnp.transpose(x, (0, 2, 1))
    outs = []
    for h in range(PH):
        vals, idx = _topk(lax.slice(s2, (h * BP, 0), ((h + 1) * BP, T)))
        rows = _gathers[h](xt, idx.reshape(BP * K))
        outs.append(
            jnp.concatenate(
                [vals.reshape(BP, K, 1), rows.reshape(BP, K, D)], axis=-1
            )
        )
    return jnp.concatenate(outs, axis=0)
